# baseline (speedup 1.0000x reference)
import dataclasses
import functools

import jax
import jax.numpy as jnp
from jax import lax
from jax.experimental import pallas as pl
from jax.experimental.pallas import tpu as pltpu
from jax.experimental.pallas import tpu_sc as plsc

NC = 2
NSUB = 16
NW = NC * NSUB
CH = 128
LANES = 16
KB = 16
DPAD = 1024


def _sc_compiler_params():
    cp = pltpu.CompilerParams()
    fields = pltpu.CompilerParams.__dataclass_fields__
    if "needs_layout_passes" in fields:
        cp = dataclasses.replace(cp, needs_layout_passes=False)
    if "use_tc_tiling_on_sc" in fields:
        cp = dataclasses.replace(cp, use_tc_tiling_on_sc=False)
    return cp


def _mm0_body(xs_ref, xt_ref, w_ref, hs_ref, ht_ref, hpk_ref):
    w = w_ref[...]
    h = lax.dot_general(
        xs_ref[...], w, (((1,), (0,)), ((), ())),
        precision=lax.Precision.HIGHEST, preferred_element_type=jnp.float32)
    hs_ref[...] = h
    hb = lax.bitcast_convert_type(h.astype(jnp.bfloat16), jnp.uint16)
    d2 = h.shape[1] // 2
    lo = hb[:, :d2].astype(jnp.uint32)
    hi = hb[:, d2:].astype(jnp.uint32)
    hpk_ref[...] = lax.bitcast_convert_type(lo | (hi << 16), jnp.int32)
    ht_ref[...] = lax.dot_general(
        xt_ref[...], w, (((1,), (0,)), ((), ())),
        precision=lax.Precision.HIGHEST, preferred_element_type=jnp.float32)


def _fin_body(hs_ref, ht_ref, acc_ref, deg_ref, w1_ref, ys_ref, yt_ref):
    w1 = w1_ref[...]
    ys_ref[...] = lax.dot_general(
        jnp.maximum(hs_ref[...], 0.0), w1, (((1,), (0,)), ((), ())),
        precision=lax.Precision.HIGHEST, preferred_element_type=jnp.float32)
    indeg = deg_ref[0, :, 0:1] + deg_ref[1, :, 0:1]
    dinv = lax.rsqrt(indeg + 1.0)
    agg = acc_ref[0] + acc_ref[1]
    z = dinv * dinv * ht_ref[...] + dinv * agg
    yt_ref[...] = lax.dot_general(
        jnp.maximum(z, 0.0), w1, (((1,), (0,)), ((), ())),
        precision=lax.Precision.HIGHEST, preferred_element_type=jnp.float32)


def _make_sc_degree(n_acc, cpt):
    rows_per_tile = n_acc // NSUB
    mesh = plsc.VectorSubcoreMesh(core_axis_name="c", subcore_axis_name="s")

    @functools.partial(
        pl.kernel,
        out_type=jax.ShapeDtypeStruct((NC, n_acc, LANES), jnp.float32),
        mesh=mesh,
        scratch_types=[
            pltpu.VMEM_SHARED((n_acc + DPAD, LANES), jnp.float32),
            pltpu.VMEM((cpt, CH), jnp.int32),
            pltpu.VMEM((CH, LANES), jnp.float32),
        ],
    )
    def sc_deg(cold_hbm, deg_out, deg_sh, col_v, ones_v):
        cid = lax.axis_index("c")
        sid = lax.axis_index("s")
        wid = cid * NSUB + sid

        @pl.loop(0, CH)
        def _(i):
            ones_v[i, pl.ds(0, LANES)] = jnp.zeros((LANES,), jnp.float32)

        base = sid * rows_per_tile
        n_full = rows_per_tile // CH
        rem = rows_per_tile - n_full * CH

        @pl.loop(0, n_full)
        def _(k):
            pltpu.sync_copy(ones_v, deg_sh.at[pl.ds(base + k * CH, CH)])

        if rem:
            pltpu.sync_copy(ones_v.at[pl.ds(0, rem)],
                            deg_sh.at[pl.ds(base + n_full * CH, rem)])

        @pl.loop(0, CH)
        def _(i):
            ones_v[i, pl.ds(0, LANES)] = jnp.ones((LANES,), jnp.float32)

        pltpu.sync_copy(cold_hbm.at[wid], col_v)
        plsc.subcore_barrier()

        @pl.loop(0, cpt)
        def _(j):
            pltpu.sync_copy(ones_v, deg_sh.at[col_v.at[j]], add=True)

        plsc.subcore_barrier()
        pltpu.sync_copy(deg_sh.at[pl.ds(base, rows_per_tile)],
                        deg_out.at[cid].at[pl.ds(base, rows_per_tile)])

    return sc_deg


def _make_sc_aggregate(n_acc, cpt, d):
    rows_per_tile = n_acc // NSUB
    nblk = cpt // KB
    dp = d // 2
    mesh = plsc.VectorSubcoreMesh(core_axis_name="c", subcore_axis_name="s")

    @functools.partial(
        pl.kernel,
        out_type=jax.ShapeDtypeStruct((NC, n_acc, d), jnp.float32),
        mesh=mesh,
        scratch_types=[
            pltpu.VMEM_SHARED((n_acc, d), jnp.float32),
            pltpu.VMEM((2, KB, CH), jnp.int32),
            pltpu.VMEM((2, KB, CH), jnp.int32),
            pltpu.VMEM((CH, dp), jnp.int32),
            pltpu.VMEM((CH, dp), jnp.int32),
            pltpu.VMEM((CH, dp), jnp.int32),
            pltpu.VMEM((CH, dp), jnp.int32),
            pltpu.VMEM((CH, d), jnp.float32),
            pltpu.VMEM((CH, d), jnp.float32),
            pltpu.SemaphoreType.DMA,
            pltpu.SemaphoreType.DMA,
            pltpu.SemaphoreType.DMA,
            pltpu.SemaphoreType.DMA,
            pltpu.SemaphoreType.DMA,
            pltpu.SemaphoreType.DMA,
            pltpu.SemaphoreType.DMA,
        ],
        compiler_params=_sc_compiler_params(),
    )
    def sc_agg(hs_hbm, rowi_hbm, coli_hbm, acc_out,
               acc_sh, row_v, col_v, gb0, gb1, gb2, gb3, sb0, sb1,
               gs0, gs1, gs2, gs3, ss0, ss1, isem):
        cid = lax.axis_index("c")
        sid = lax.axis_index("s")
        wid = cid * NSUB + sid
        gbufs = (gb0, gb1, gb2, gb3)
        sbufs = (sb0, sb1)
        gsems = (gs0, gs1, gs2, gs3)
        ssems = (ss0, ss1)

        @pl.loop(0, CH)
        def _(i):
            @pl.loop(0, d, step=LANES)
            def _(j):
                sb0[i, pl.ds(j, LANES)] = jnp.zeros((LANES,), jnp.float32)

        base = sid * rows_per_tile
        n_full = rows_per_tile // CH
        rem = rows_per_tile - n_full * CH

        @pl.loop(0, n_full)
        def _(k):
            pltpu.sync_copy(sb0, acc_sh.at[pl.ds(base + k * CH, CH)])

        if rem:
            pltpu.sync_copy(sb0.at[pl.ds(0, rem)],
                            acc_sh.at[pl.ds(base + n_full * CH, rem)])

        plsc.subcore_barrier()

        def unpack_chunk(gb, sb):
            @pl.loop(0, CH, step=4)
            def _(i):
                for r in range(4):
                    for j in range(0, dp, LANES):
                        w = gb[i + r, pl.ds(j, LANES)]
                        lo = plsc.bitcast(w << 16, jnp.float32)
                        hi = plsc.bitcast(w & jnp.int32(-65536), jnp.float32)
                        sb[i + r, pl.ds(j, LANES)] = lo
                        sb[i + r, pl.ds(dp + j, LANES)] = hi

        pltpu.sync_copy(rowi_hbm.at[wid].at[pl.ds(0, KB)], row_v.at[0])
        pltpu.sync_copy(coli_hbm.at[wid].at[pl.ds(0, KB)], col_v.at[0])
        if nblk > 1:
            pltpu.async_copy(rowi_hbm.at[wid].at[pl.ds(KB, KB)],
                             row_v.at[1], isem)
            pltpu.async_copy(coli_hbm.at[wid].at[pl.ds(KB, KB)],
                             col_v.at[1], isem)

        @pl.loop(0, nblk)
        def _(b):
            sl = b % 2
            rv = row_v.at[sl]
            cv = col_v.at[sl]

            @pl.when(b > 0)
            def _():
                pltpu.make_async_copy(rowi_hbm.at[wid].at[pl.ds(b * KB, KB)],
                                      row_v.at[sl], isem).wait()
                pltpu.make_async_copy(coli_hbm.at[wid].at[pl.ds(b * KB, KB)],
                                      col_v.at[sl], isem).wait()

            @pl.when(b + 1 < nblk)
            def _():
                nsl = (b + 1) % 2
                pltpu.async_copy(
                    rowi_hbm.at[wid].at[pl.ds((b + 1) * KB, KB)],
                    row_v.at[nsl], isem)
                pltpu.async_copy(
                    coli_hbm.at[wid].at[pl.ds((b + 1) * KB, KB)],
                    col_v.at[nsl], isem)

            pltpu.async_copy(hs_hbm.at[rv.at[0]], gb0, gs0)
            pltpu.async_copy(hs_hbm.at[rv.at[1]], gb1, gs1)

            @pl.loop(0, KB, step=4)
            def _(j):
                for t in range(4):
                    c = j + t
                    gb, gt = gbufs[t], gsems[t]
                    g2, gt2 = gbufs[(t + 2) % 4], gsems[(t + 2) % 4]
                    sb, st = sbufs[t % 2], ssems[t % 2]
                    pltpu.make_async_copy(hs_hbm.at[rv.at[c]], gb,
                                          gt).wait()

                    @pl.when(c + 2 < KB)
                    def _():
                        pltpu.async_copy(hs_hbm.at[rv.at[c + 2]], g2, gt2)

                    @pl.when(c >= 2)
                    def _():
                        pltpu.make_async_copy(
                            sb, acc_sh.at[cv.at[c - 2]], st).wait()

                    unpack_chunk(gb, sb)
                    pltpu.async_copy(sb, acc_sh.at[cv.at[c]], st, add=True)

            pltpu.make_async_copy(sb0, acc_sh.at[cv.at[KB - 2]],
                                  ss0).wait()
            pltpu.make_async_copy(sb1, acc_sh.at[cv.at[KB - 1]],
                                  ss1).wait()

        plsc.subcore_barrier()

        pltpu.sync_copy(acc_sh.at[pl.ds(base, rows_per_tile)],
                        acc_out.at[cid].at[pl.ds(base, rows_per_tile)])

    return sc_agg


def _impl(edge_index, x_s, x_t, W0, W1):
    n_s, d_in = x_s.shape
    n_t = x_t.shape[0]
    d_hid = W0.shape[1]
    d_out = W1.shape[1]
    e = edge_index.shape[1]

    row = edge_index[0].astype(jnp.int32)
    col = edge_index[1].astype(jnp.int32)

    cpt = -(-e // (NW * CH))
    cpt = -(-cpt // KB) * KB
    e_pad = NW * cpt * CH
    n_acc = -(-(n_t + 1) // (NSUB * 8)) * (NSUB * 8)
    n_sp = -(-(n_s + 1) // 8) * 8

    npad = e_pad - e
    pad_iota = lax.iota(jnp.int32, npad)
    row_p = jnp.concatenate([row, jnp.full((npad,), n_s, jnp.int32)])
    col_p = jnp.concatenate([col, pad_iota % n_t])
    col_d = jnp.concatenate([col, n_acc + pad_iota % DPAD])
    row_p = row_p.reshape(cpt, NW, CH).transpose(1, 0, 2)
    col_p = col_p.reshape(cpt, NW, CH).transpose(1, 0, 2)
    col_d = col_d.reshape(cpt, NW, CH).transpose(1, 0, 2)

    sc_deg = _make_sc_degree(n_acc, cpt)
    deg = sc_deg(col_d)

    xs_p = jnp.pad(x_s, ((0, n_sp - n_s), (0, 0)))
    hs_p, ht, hs_pk = pl.pallas_call(
        _mm0_body,
        out_shape=(
            jax.ShapeDtypeStruct((n_sp, d_hid), jnp.float32),
            jax.ShapeDtypeStruct((n_t, d_hid), jnp.float32),
            jax.ShapeDtypeStruct((n_sp, d_hid // 2), jnp.int32),
        ),
    )(xs_p, x_t, W0)

    sc_agg = _make_sc_aggregate(n_acc, cpt, d_hid)
    acc = sc_agg(hs_pk, row_p, col_p)

    ys, yt = pl.pallas_call(
        _fin_body,
        out_shape=(
            jax.ShapeDtypeStruct((n_s, d_out), jnp.float32),
            jax.ShapeDtypeStruct((n_t, d_out), jnp.float32),
        ),
        grid=(1,),
        in_specs=[
            pl.BlockSpec((n_s, d_hid), lambda i: (0, 0)),
            pl.BlockSpec((n_t, d_hid), lambda i: (0, 0)),
            pl.BlockSpec((NC, n_t, d_hid), lambda i: (0, 0, 0)),
            pl.BlockSpec((NC, n_t, LANES), lambda i: (0, 0, 0)),
            pl.BlockSpec((d_hid, d_out), lambda i: (0, 0)),
        ],
        out_specs=(
            pl.BlockSpec((n_s, d_out), lambda i: (0, 0)),
            pl.BlockSpec((n_t, d_out), lambda i: (0, 0)),
        ),
    )(hs_p, ht, acc, deg, W1)

    return ys, yt


_impl_jit = jax.jit(_impl)


def kernel(edge_index, x_s, x_t, W0, W1):
    return _impl_jit(edge_index, x_s, x_t, W0, W1)


# speedup vs baseline: 1.0067x; 1.0067x over previous
import dataclasses
import functools

import jax
import jax.numpy as jnp
from jax import lax
from jax.experimental import pallas as pl
from jax.experimental.pallas import tpu as pltpu
from jax.experimental.pallas import tpu_sc as plsc

NC = 2
NSUB = 16
NW = NC * NSUB
CH = 128
LANES = 16
KB = 16
DPAD = 1024


def _sc_compiler_params():
    cp = pltpu.CompilerParams()
    fields = pltpu.CompilerParams.__dataclass_fields__
    if "needs_layout_passes" in fields:
        cp = dataclasses.replace(cp, needs_layout_passes=False)
    if "use_tc_tiling_on_sc" in fields:
        cp = dataclasses.replace(cp, use_tc_tiling_on_sc=False)
    return cp


def _mm0_body(xs_ref, xt_ref, w_ref, hs_ref, ht_ref, hpk_ref):
    w = w_ref[...]
    h = lax.dot_general(
        xs_ref[...], w, (((1,), (0,)), ((), ())),
        precision=lax.Precision.HIGHEST, preferred_element_type=jnp.float32)
    hs_ref[...] = h
    hb = lax.bitcast_convert_type(h.astype(jnp.bfloat16), jnp.uint16)
    d2 = h.shape[1] // 2
    lo = hb[:, :d2].astype(jnp.uint32)
    hi = hb[:, d2:].astype(jnp.uint32)
    hpk_ref[...] = lax.bitcast_convert_type(lo | (hi << 16), jnp.int32)
    ht_ref[...] = lax.dot_general(
        xt_ref[...], w, (((1,), (0,)), ((), ())),
        precision=lax.Precision.HIGHEST, preferred_element_type=jnp.float32)


def _fin_body(hs_ref, ht_ref, acc_ref, deg_ref, w1_ref, ys_ref, yt_ref):
    w1 = w1_ref[...]
    ys_ref[...] = lax.dot_general(
        jnp.maximum(hs_ref[...], 0.0), w1, (((1,), (0,)), ((), ())),
        precision=lax.Precision.HIGHEST, preferred_element_type=jnp.float32)
    indeg = deg_ref[0, :, 0:1] + deg_ref[1, :, 0:1]
    dinv = lax.rsqrt(indeg + 1.0)
    agg = acc_ref[0] + acc_ref[1]
    z = dinv * dinv * ht_ref[...] + dinv * agg
    yt_ref[...] = lax.dot_general(
        jnp.maximum(z, 0.0), w1, (((1,), (0,)), ((), ())),
        precision=lax.Precision.HIGHEST, preferred_element_type=jnp.float32)


def _make_sc_degree(n_acc, cpt):
    rows_per_tile = n_acc // NSUB
    mesh = plsc.VectorSubcoreMesh(core_axis_name="c", subcore_axis_name="s")

    @functools.partial(
        pl.kernel,
        out_type=jax.ShapeDtypeStruct((NC, n_acc, LANES), jnp.float32),
        mesh=mesh,
        scratch_types=[
            pltpu.VMEM_SHARED((n_acc + DPAD, LANES), jnp.float32),
            pltpu.VMEM((cpt, CH), jnp.int32),
            pltpu.VMEM((CH, LANES), jnp.float32),
        ],
    )
    def sc_deg(cold_hbm, deg_out, deg_sh, col_v, ones_v):
        cid = lax.axis_index("c")
        sid = lax.axis_index("s")
        wid = cid * NSUB + sid

        @pl.loop(0, CH)
        def _(i):
            ones_v[i, pl.ds(0, LANES)] = jnp.zeros((LANES,), jnp.float32)

        base = sid * rows_per_tile
        n_full = rows_per_tile // CH
        rem = rows_per_tile - n_full * CH

        @pl.loop(0, n_full)
        def _(k):
            pltpu.sync_copy(ones_v, deg_sh.at[pl.ds(base + k * CH, CH)])

        if rem:
            pltpu.sync_copy(ones_v.at[pl.ds(0, rem)],
                            deg_sh.at[pl.ds(base + n_full * CH, rem)])

        @pl.loop(0, CH)
        def _(i):
            ones_v[i, pl.ds(0, LANES)] = jnp.ones((LANES,), jnp.float32)

        pltpu.sync_copy(cold_hbm.at[wid], col_v)
        plsc.subcore_barrier()

        @pl.loop(0, cpt)
        def _(j):
            pltpu.sync_copy(ones_v, deg_sh.at[col_v.at[j]], add=True)

        plsc.subcore_barrier()
        pltpu.sync_copy(deg_sh.at[pl.ds(base, rows_per_tile)],
                        deg_out.at[cid].at[pl.ds(base, rows_per_tile)])

    return sc_deg


def _make_sc_aggregate(n_acc, cpt, d):
    rows_per_tile = n_acc // NSUB
    nblk = cpt // KB
    dp = d // 2
    mesh = plsc.VectorSubcoreMesh(core_axis_name="c", subcore_axis_name="s")

    @functools.partial(
        pl.kernel,
        out_type=jax.ShapeDtypeStruct((NC, n_acc, d), jnp.float32),
        mesh=mesh,
        scratch_types=[
            pltpu.VMEM_SHARED((n_acc, d), jnp.float32),
            pltpu.VMEM((2, KB, CH), jnp.int32),
            pltpu.VMEM((2, KB, CH), jnp.int32),
            pltpu.VMEM((CH, dp), jnp.int32),
            pltpu.VMEM((CH, dp), jnp.int32),
            pltpu.VMEM((CH, dp), jnp.int32),
            pltpu.VMEM((CH, dp), jnp.int32),
            pltpu.VMEM((CH, d), jnp.float32),
            pltpu.VMEM((CH, d), jnp.float32),
            pltpu.SemaphoreType.DMA,
            pltpu.SemaphoreType.DMA,
            pltpu.SemaphoreType.DMA,
            pltpu.SemaphoreType.DMA,
            pltpu.SemaphoreType.DMA,
            pltpu.SemaphoreType.DMA,
            pltpu.SemaphoreType.DMA,
        ],
        compiler_params=_sc_compiler_params(),
    )
    def sc_agg(hs_hbm, rowi_hbm, coli_hbm, acc_out,
               acc_sh, row_v, col_v, gb0, gb1, gb2, gb3, sb0, sb1,
               gs0, gs1, gs2, gs3, ss0, ss1, isem):
        cid = lax.axis_index("c")
        sid = lax.axis_index("s")
        wid = cid * NSUB + sid
        gbufs = (gb0, gb1, gb2, gb3)
        sbufs = (sb0, sb1)
        gsems = (gs0, gs1, gs2, gs3)
        ssems = (ss0, ss1)

        @pl.loop(0, CH)
        def _(i):
            @pl.loop(0, d, step=LANES)
            def _(j):
                sb0[i, pl.ds(j, LANES)] = jnp.zeros((LANES,), jnp.float32)

        base = sid * rows_per_tile
        n_full = rows_per_tile // CH
        rem = rows_per_tile - n_full * CH

        @pl.loop(0, n_full)
        def _(k):
            pltpu.sync_copy(sb0, acc_sh.at[pl.ds(base + k * CH, CH)])

        if rem:
            pltpu.sync_copy(sb0.at[pl.ds(0, rem)],
                            acc_sh.at[pl.ds(base + n_full * CH, rem)])

        plsc.subcore_barrier()

        def unpack_chunk(gb, sb):
            @pl.loop(0, CH)
            def _(i):
                for j in range(0, dp, LANES):
                    w = gb[i, pl.ds(j, LANES)]
                    lo = plsc.bitcast(w << 16, jnp.float32)
                    hi = plsc.bitcast(w & jnp.int32(-65536), jnp.float32)
                    sb[i, pl.ds(j, LANES)] = lo
                    sb[i, pl.ds(dp + j, LANES)] = hi

        pltpu.sync_copy(rowi_hbm.at[wid].at[pl.ds(0, KB)], row_v.at[0])
        pltpu.sync_copy(coli_hbm.at[wid].at[pl.ds(0, KB)], col_v.at[0])
        if nblk > 1:
            pltpu.async_copy(rowi_hbm.at[wid].at[pl.ds(KB, KB)],
                             row_v.at[1], isem)
            pltpu.async_copy(coli_hbm.at[wid].at[pl.ds(KB, KB)],
                             col_v.at[1], isem)

        @pl.loop(0, nblk)
        def _(b):
            sl = b % 2
            rv = row_v.at[sl]
            cv = col_v.at[sl]

            @pl.when(b > 0)
            def _():
                pltpu.make_async_copy(rowi_hbm.at[wid].at[pl.ds(b * KB, KB)],
                                      row_v.at[sl], isem).wait()
                pltpu.make_async_copy(coli_hbm.at[wid].at[pl.ds(b * KB, KB)],
                                      col_v.at[sl], isem).wait()

            @pl.when(b + 1 < nblk)
            def _():
                nsl = (b + 1) % 2
                pltpu.async_copy(
                    rowi_hbm.at[wid].at[pl.ds((b + 1) * KB, KB)],
                    row_v.at[nsl], isem)
                pltpu.async_copy(
                    coli_hbm.at[wid].at[pl.ds((b + 1) * KB, KB)],
                    col_v.at[nsl], isem)

            pltpu.async_copy(hs_hbm.at[rv.at[0]], gb0, gs0)
            pltpu.async_copy(hs_hbm.at[rv.at[1]], gb1, gs1)

            @pl.loop(0, KB, step=4)
            def _(j):
                for t in range(4):
                    c = j + t
                    gb, gt = gbufs[t], gsems[t]
                    g2, gt2 = gbufs[(t + 2) % 4], gsems[(t + 2) % 4]
                    sb, st = sbufs[t % 2], ssems[t % 2]
                    pltpu.make_async_copy(hs_hbm.at[rv.at[c]], gb,
                                          gt).wait()

                    @pl.when(c + 2 < KB)
                    def _():
                        pltpu.async_copy(hs_hbm.at[rv.at[c + 2]], g2, gt2)

                    @pl.when(c >= 2)
                    def _():
                        pltpu.make_async_copy(
                            sb, acc_sh.at[cv.at[c - 2]], st).wait()

                    unpack_chunk(gb, sb)
                    pltpu.async_copy(sb, acc_sh.at[cv.at[c]], st, add=True)

            pltpu.make_async_copy(sb0, acc_sh.at[cv.at[KB - 2]],
                                  ss0).wait()
            pltpu.make_async_copy(sb1, acc_sh.at[cv.at[KB - 1]],
                                  ss1).wait()

        plsc.subcore_barrier()

        pltpu.sync_copy(acc_sh.at[pl.ds(base, rows_per_tile)],
                        acc_out.at[cid].at[pl.ds(base, rows_per_tile)])

    return sc_agg


def _impl(edge_index, x_s, x_t, W0, W1):
    n_s, d_in = x_s.shape
    n_t = x_t.shape[0]
    d_hid = W0.shape[1]
    d_out = W1.shape[1]
    e = edge_index.shape[1]

    row = edge_index[0].astype(jnp.int32)
    col = edge_index[1].astype(jnp.int32)

    cpt = -(-e // (NW * CH))
    cpt = -(-cpt // KB) * KB
    e_pad = NW * cpt * CH
    n_acc = -(-(n_t + 1) // (NSUB * 8)) * (NSUB * 8)
    n_sp = -(-(n_s + 1) // 8) * 8

    npad = e_pad - e
    pad_iota = lax.iota(jnp.int32, npad)
    row_p = jnp.concatenate([row, jnp.full((npad,), n_s, jnp.int32)])
    col_p = jnp.concatenate([col, pad_iota % n_t])
    col_d = jnp.concatenate([col, n_acc + pad_iota % DPAD])
    row_p = row_p.reshape(cpt, NW, CH).transpose(1, 0, 2)
    col_p = col_p.reshape(cpt, NW, CH).transpose(1, 0, 2)
    col_d = col_d.reshape(cpt, NW, CH).transpose(1, 0, 2)

    sc_deg = _make_sc_degree(n_acc, cpt)
    deg = sc_deg(col_d)

    xs_p = jnp.pad(x_s, ((0, n_sp - n_s), (0, 0)))
    hs_p, ht, hs_pk = pl.pallas_call(
        _mm0_body,
        out_shape=(
            jax.ShapeDtypeStruct((n_sp, d_hid), jnp.float32),
            jax.ShapeDtypeStruct((n_t, d_hid), jnp.float32),
            jax.ShapeDtypeStruct((n_sp, d_hid // 2), jnp.int32),
        ),
    )(xs_p, x_t, W0)

    sc_agg = _make_sc_aggregate(n_acc, cpt, d_hid)
    acc = sc_agg(hs_pk, row_p, col_p)

    ys, yt = pl.pallas_call(
        _fin_body,
        out_shape=(
            jax.ShapeDtypeStruct((n_s, d_out), jnp.float32),
            jax.ShapeDtypeStruct((n_t, d_out), jnp.float32),
        ),
        grid=(1,),
        in_specs=[
            pl.BlockSpec((n_s, d_hid), lambda i: (0, 0)),
            pl.BlockSpec((n_t, d_hid), lambda i: (0, 0)),
            pl.BlockSpec((NC, n_t, d_hid), lambda i: (0, 0, 0)),
            pl.BlockSpec((NC, n_t, LANES), lambda i: (0, 0, 0)),
            pl.BlockSpec((d_hid, d_out), lambda i: (0, 0)),
        ],
        out_specs=(
            pl.BlockSpec((n_s, d_out), lambda i: (0, 0)),
            pl.BlockSpec((n_t, d_out), lambda i: (0, 0)),
        ),
    )(hs_p, ht, acc, deg, W1)

    return ys, yt


_impl_jit = jax.jit(_impl)


def kernel(edge_index, x_s, x_t, W0, W1):
    return _impl_jit(edge_index, x_s, x_t, W0, W1)
